# Initial kernel scaffold; baseline (speedup 1.0000x reference)
import jax
import jax.numpy as jnp
from jax import lax
from jax.experimental import pallas as pl
from jax.experimental.pallas import tpu as pltpu
from jax.experimental.pallas import tpu_sc as plsc

N = 10000
E = 160000
D = 512
DO = 256
FC = 128
NK = D // FC
ER = 1280
EROWS_VALID = E // 128
NPAD = 10240
RB = 1000

NUM_CORES = 2
NUM_SUBCORES = 16
ROWS_PER_TEC = ER // (NUM_CORES * NUM_SUBCORES)
NSLICE = NPAD // NUM_SUBCORES

_MESH = plsc.VectorSubcoreMesh(
    core_axis_name="c", subcore_axis_name="s",
    num_cores=NUM_CORES, num_subcores=NUM_SUBCORES)


def _k1_body(feat_ref, w1_ref, att2_ref, xflat_ref, a2_ref):
    k = pl.program_id(1)
    xk = jnp.dot(feat_ref[...], w1_ref[...], preferred_element_type=jnp.float32)
    xflat_ref[...] = xk

    @pl.when(k == 0)
    def _():
        a2_ref[...] = jnp.zeros_like(a2_ref)

    a2_ref[...] += jnp.dot(xk, att2_ref[...], preferred_element_type=jnp.float32)


_k1 = pl.pallas_call(
    _k1_body,
    grid=(N // RB, NK),
    in_specs=[
        pl.BlockSpec((RB, D), lambda i, k: (i, 0)),
        pl.BlockSpec((D, FC), lambda i, k: (0, k)),
        pl.BlockSpec((FC, 128), lambda i, k: (k, 0)),
    ],
    out_specs=[
        pl.BlockSpec((RB, FC), lambda i, k: (k * (N // RB) + i, 0)),
        pl.BlockSpec((RB, 128), lambda i, k: (i, 0)),
    ],
    out_shape=[
        jax.ShapeDtypeStruct((NK * N, FC), jnp.float32),
        jax.ShapeDtypeStruct((N, 128), jnp.float32),
    ],
)


def _elu(x):
    return jnp.where(x > 0, x, jnp.exp(x) - 1.0)


def _k2_body(p_ref, d2_ref, w2_ref, w2t_ref, h2_ref, x3_ref):
    rd = 1.0 / (d2_ref[0] + d2_ref[1] + 1e-16)
    acc = jnp.zeros((RB, DO), jnp.float32)
    for k in range(NK):
        p = (p_ref[0, k] + p_ref[1, k]) * rd[:, None]
        h = _elu(p)
        acc += jnp.dot(h, w2_ref[k * FC:(k + 1) * FC, :],
                       preferred_element_type=jnp.float32)
    h2_ref[...] = acc
    for k in range(NK):
        x3_ref[k] = jnp.dot(acc, w2t_ref[:, k * FC:(k + 1) * FC],
                            preferred_element_type=jnp.float32)


_k2 = pl.pallas_call(
    _k2_body,
    grid=(N // RB,),
    in_specs=[
        pl.BlockSpec((NUM_CORES, NK, RB, FC), lambda i: (0, 0, i, 0)),
        pl.BlockSpec((NUM_CORES, RB), lambda i: (0, i)),
        pl.BlockSpec((D, DO), lambda i: (0, 0)),
        pl.BlockSpec((DO, D), lambda i: (0, 0)),
    ],
    out_specs=[
        pl.BlockSpec((RB, DO), lambda i: (i, 0)),
        pl.BlockSpec((NK, RB, FC), lambda i: (0, i, 0)),
    ],
    out_shape=[
        jax.ShapeDtypeStruct((N, DO), jnp.float32),
        jax.ShapeDtypeStruct((NK, N, FC), jnp.float32),
    ],
)


def _k3_body(p_ref, d2_ref, w1t_ref, h4_ref):
    rd = 1.0 / (d2_ref[0] + d2_ref[1] + 1e-16)
    acc = jnp.zeros((RB, D), jnp.float32)
    for k in range(NK):
        p = (p_ref[0, k] + p_ref[1, k]) * rd[:, None]
        h = _elu(p)
        acc += jnp.dot(h, w1t_ref[k * FC:(k + 1) * FC, :],
                       preferred_element_type=jnp.float32)
    h4_ref[...] = acc


_k3 = pl.pallas_call(
    _k3_body,
    grid=(N // RB,),
    in_specs=[
        pl.BlockSpec((NUM_CORES, NK, RB, FC), lambda i: (0, 0, i, 0)),
        pl.BlockSpec((NUM_CORES, RB), lambda i: (0, i)),
        pl.BlockSpec((D, D), lambda i: (0, 0)),
    ],
    out_specs=pl.BlockSpec((RB, D), lambda i: (i, 0)),
    out_shape=jax.ShapeDtypeStruct((N, D), jnp.float32),
)


def _edge_w_body(asrc_hbm, adst_hbm, srcr_hbm, dstr_hbm, exr_hbm, den2_hbm,
                 asrc_v, adst_v, src_v, dst_v, ex_v, zb_v, den_sp):
    c = lax.axis_index("c")
    t = lax.axis_index("s")
    rowbase = c * (NUM_SUBCORES * ROWS_PER_TEC) + t * ROWS_PER_TEC

    pltpu.sync_copy(asrc_hbm, asrc_v)
    pltpu.sync_copy(adst_hbm, adst_v)
    pltpu.sync_copy(srcr_hbm.at[pl.ds(rowbase, ROWS_PER_TEC)], src_v)
    pltpu.sync_copy(dstr_hbm.at[pl.ds(rowbase, ROWS_PER_TEC)], dst_v)

    def _zb(i, carry):
        zb_v[pl.ds(i * 16, 16)] = jnp.zeros((16,), jnp.float32)
        return carry
    lax.fori_loop(0, NSLICE // 16, _zb, 0)
    pltpu.sync_copy(zb_v, den_sp.at[pl.ds(t * NSLICE, NSLICE)])

    def _row(b, carry):
        valid = (rowbase + b) < EROWS_VALID

        def _grp(j, carry2):
            sl = pl.ds(j * 16, 16)
            si = src_v[b, sl]
            di = dst_v[b, sl]
            av = plsc.load_gather(asrc_v, [si])
            bv = plsc.load_gather(adst_v, [di])
            alpha = 1.0 / (1.0 + jnp.exp(-(av + bv)))
            ex = jnp.exp(alpha)
            ex = jnp.where(valid, ex, jnp.zeros((16,), jnp.float32))
            ex_v[b, sl] = ex
            return carry2
        lax.fori_loop(0, 8, _grp, 0)
        return carry
    lax.fori_loop(0, ROWS_PER_TEC, _row, 0)

    pltpu.sync_copy(ex_v, exr_hbm.at[pl.ds(rowbase, ROWS_PER_TEC)])

    plsc.subcore_barrier()

    def _scat(b, carry):
        pltpu.sync_copy(ex_v.at[b], den_sp.at[dst_v.at[b]], add=True)
        return carry
    lax.fori_loop(0, ROWS_PER_TEC, _scat, 0)

    plsc.subcore_barrier()
    pltpu.sync_copy(den_sp.at[pl.ds(t * NSLICE, NSLICE)],
                    den2_hbm.at[c, pl.ds(t * NSLICE, NSLICE)])


_edge_w = pl.kernel(
    _edge_w_body,
    out_type=[
        jax.ShapeDtypeStruct((ER, 128), jnp.float32),
        jax.ShapeDtypeStruct((NUM_CORES, NPAD), jnp.float32),
    ],
    mesh=_MESH,
    scratch_types=[
        pltpu.VMEM((N,), jnp.float32),
        pltpu.VMEM((N,), jnp.float32),
        pltpu.VMEM((ROWS_PER_TEC, 128), jnp.int32),
        pltpu.VMEM((ROWS_PER_TEC, 128), jnp.int32),
        pltpu.VMEM((ROWS_PER_TEC, 128), jnp.float32),
        pltpu.VMEM((NSLICE,), jnp.float32),
        pltpu.VMEM_SHARED((NPAD,), jnp.float32),
    ],
)


def _spmm_body(xflat_hbm, srcr_hbm, dstr_hbm, exr_hbm, out_hbm,
               src_v, dst_v, ex_v, idx_v, rows_v, zb_v, acc_sp, sem):
    c = lax.axis_index("c")
    t = lax.axis_index("s")
    rowbase = c * (NUM_SUBCORES * ROWS_PER_TEC) + t * ROWS_PER_TEC

    pltpu.sync_copy(srcr_hbm.at[pl.ds(rowbase, ROWS_PER_TEC)], src_v)
    pltpu.sync_copy(dstr_hbm.at[pl.ds(rowbase, ROWS_PER_TEC)], dst_v)
    pltpu.sync_copy(exr_hbm.at[pl.ds(rowbase, ROWS_PER_TEC)], ex_v)

    def _zr(r, carry):
        def _zj(j, carry2):
            zb_v[r, pl.ds(j * 16, 16)] = jnp.zeros((16,), jnp.float32)
            return carry2
        lax.fori_loop(0, FC // 16, _zj, 0)
        return carry
    lax.fori_loop(0, 128, _zr, 0)

    def _chunk(k, carry):
        def _z(i, carry2):
            pltpu.sync_copy(zb_v, acc_sp.at[pl.ds(t * NSLICE + i * 128, 128)])
            return carry2
        lax.fori_loop(0, NSLICE // 128, _z, 0)

        base = k * N

        def _bi(b, carry2):
            def _bj(j, carry3):
                sl = pl.ds(j * 16, 16)
                idx_v[b, sl] = src_v[b, sl] + base
                return carry3
            lax.fori_loop(0, 8, _bj, 0)
            return carry2
        lax.fori_loop(0, ROWS_PER_TEC, _bi, 0)

        plsc.subcore_barrier()

        def _edge_batch(b, carry2):
            pltpu.async_copy(xflat_hbm.at[idx_v.at[b]], rows_v, sem).wait()

            def _scale_row(r, carry3):
                w = ex_v[b, r]

                def _cj(j, carry4):
                    sl = pl.ds(j * 16, 16)
                    rows_v[r, sl] = rows_v[r, sl] * w
                    return carry4
                lax.fori_loop(0, FC // 16, _cj, 0)
                return carry3
            lax.fori_loop(0, 128, _scale_row, 0)

            pltpu.sync_copy(rows_v, acc_sp.at[dst_v.at[b]], add=True)
            return carry2
        lax.fori_loop(0, ROWS_PER_TEC, _edge_batch, 0)

        plsc.subcore_barrier()
        pltpu.sync_copy(acc_sp.at[pl.ds(t * NSLICE, NSLICE)],
                        out_hbm.at[c, k, pl.ds(t * NSLICE, NSLICE)])
        return carry
    lax.fori_loop(0, NK, _chunk, 0)


_spmm = pl.kernel(
    _spmm_body,
    out_type=jax.ShapeDtypeStruct((NUM_CORES, NK, NPAD, FC), jnp.float32),
    mesh=_MESH,
    scratch_types=[
        pltpu.VMEM((ROWS_PER_TEC, 128), jnp.int32),
        pltpu.VMEM((ROWS_PER_TEC, 128), jnp.int32),
        pltpu.VMEM((ROWS_PER_TEC, 128), jnp.float32),
        pltpu.VMEM((ROWS_PER_TEC, 128), jnp.int32),
        pltpu.VMEM((128, FC), jnp.float32),
        pltpu.VMEM((128, FC), jnp.float32),
        pltpu.VMEM_SHARED((NPAD, FC), jnp.float32),
        pltpu.SemaphoreType.DMA,
    ],
)


def kernel(features, edge_index, W1, att_src1, att_dst1, W2):
    src = edge_index[0]
    dst = edge_index[1]
    pad = ER * 128 - E
    zpad = jnp.zeros((pad,), jnp.int32)
    srcr = jnp.concatenate([src, zpad]).reshape(ER, 128)
    dstr = jnp.concatenate([dst, zpad]).reshape(ER, 128)
    att2 = (jnp.zeros((D, 128), jnp.float32)
            .at[:, 0].set(att_src1).at[:, 1].set(att_dst1))

    xflat, a2 = _k1(features, W1, att2)
    a_src = a2[:, 0]
    a_dst = a2[:, 1]

    exr, den2 = _edge_w(a_src, a_dst, srcr, dstr)

    p1 = _spmm(xflat, srcr, dstr, exr)
    h2, x3 = _k2(p1, den2, W2, W2.T)

    p2 = _spmm(x3.reshape(NK * N, FC), srcr, dstr, exr)
    h4 = _k3(p2, den2, W1.T)
    return (h2, h4)


# speedup vs baseline: 2.7736x; 2.7736x over previous
import jax
import jax.numpy as jnp
from jax import lax
from jax.experimental import pallas as pl
from jax.experimental.pallas import tpu as pltpu
from jax.experimental.pallas import tpu_sc as plsc

N = 10000
E = 160000
D = 512
DO = 256
FC = 128
NK = D // FC
ER = 1280
EROWS_VALID = E // 128
NPAD = 10240
RB = 1000

NUM_CORES = 2
NUM_SUBCORES = 16
ROWS_PER_TEC = ER // (NUM_CORES * NUM_SUBCORES)
NSLICE = NPAD // NUM_SUBCORES

_MESH = plsc.VectorSubcoreMesh(
    core_axis_name="c", subcore_axis_name="s",
    num_cores=NUM_CORES, num_subcores=NUM_SUBCORES)

_SC_PARAMS = pltpu.CompilerParams(needs_layout_passes=False)


def _k1_body(feat_ref, w1_ref, att2_ref, xflat_ref, a2_ref):
    k = pl.program_id(1)
    xk = jnp.dot(feat_ref[...], w1_ref[...], preferred_element_type=jnp.float32)
    xflat_ref[...] = xk

    @pl.when(k == 0)
    def _():
        a2_ref[...] = jnp.zeros_like(a2_ref)

    a2_ref[...] += jnp.dot(xk, att2_ref[...], preferred_element_type=jnp.float32)


_k1 = pl.pallas_call(
    _k1_body,
    grid=(N // RB, NK),
    in_specs=[
        pl.BlockSpec((RB, D), lambda i, k: (i, 0)),
        pl.BlockSpec((D, FC), lambda i, k: (0, k)),
        pl.BlockSpec((FC, 128), lambda i, k: (k, 0)),
    ],
    out_specs=[
        pl.BlockSpec((RB, FC), lambda i, k: (k * (N // RB) + i, 0)),
        pl.BlockSpec((RB, 128), lambda i, k: (i, 0)),
    ],
    out_shape=[
        jax.ShapeDtypeStruct((NK * N, FC), jnp.float32),
        jax.ShapeDtypeStruct((N, 128), jnp.float32),
    ],
)


def _elu(x):
    return jnp.where(x > 0, x, jnp.exp(x) - 1.0)


def _k2_body(p_ref, d2_ref, w2_ref, w2t_ref, h2_ref, x3_ref):
    rd = 1.0 / (d2_ref[0, 0] + 1e-16)
    acc = jnp.zeros((RB, DO), jnp.float32)
    for k in range(NK):
        p = (p_ref[0, k] + p_ref[1, k]) * rd[:, None]
        h = _elu(p)
        acc += jnp.dot(h, w2_ref[k * FC:(k + 1) * FC, :],
                       preferred_element_type=jnp.float32)
    h2_ref[...] = acc
    for k in range(NK):
        x3_ref[k] = jnp.dot(acc, w2t_ref[:, k * FC:(k + 1) * FC],
                            preferred_element_type=jnp.float32)


_k2 = pl.pallas_call(
    _k2_body,
    grid=(N // RB,),
    in_specs=[
        pl.BlockSpec((NUM_CORES, NK, RB, FC), lambda i: (0, 0, i, 0)),
        pl.BlockSpec((1, 1, RB), lambda i: (i, 0, 0)),
        pl.BlockSpec((D, DO), lambda i: (0, 0)),
        pl.BlockSpec((DO, D), lambda i: (0, 0)),
    ],
    out_specs=[
        pl.BlockSpec((RB, DO), lambda i: (i, 0)),
        pl.BlockSpec((NK, RB, FC), lambda i: (0, i, 0)),
    ],
    out_shape=[
        jax.ShapeDtypeStruct((N, DO), jnp.float32),
        jax.ShapeDtypeStruct((NK, N, FC), jnp.float32),
    ],
)


def _k3_body(p_ref, d2_ref, w1t_ref, h4_ref):
    rd = 1.0 / (d2_ref[0, 0] + 1e-16)
    acc = jnp.zeros((RB, D), jnp.float32)
    for k in range(NK):
        p = (p_ref[0, k] + p_ref[1, k]) * rd[:, None]
        h = _elu(p)
        acc += jnp.dot(h, w1t_ref[k * FC:(k + 1) * FC, :],
                       preferred_element_type=jnp.float32)
    h4_ref[...] = acc


_k3 = pl.pallas_call(
    _k3_body,
    grid=(N // RB,),
    in_specs=[
        pl.BlockSpec((NUM_CORES, NK, RB, FC), lambda i: (0, 0, i, 0)),
        pl.BlockSpec((1, 1, RB), lambda i: (i, 0, 0)),
        pl.BlockSpec((D, D), lambda i: (0, 0)),
    ],
    out_specs=pl.BlockSpec((RB, D), lambda i: (i, 0)),
    out_shape=jax.ShapeDtypeStruct((N, D), jnp.float32),
)


def _edge_w_body(asrc_hbm, adst_hbm, srcr_hbm, dstr_hbm, exr_hbm, den2_hbm,
                 asrc_v, adst_v, src_v, dst_v, ex_v, zb_v, den_sp):
    c = lax.axis_index("c")
    t = lax.axis_index("s")
    rowbase = c * (NUM_SUBCORES * ROWS_PER_TEC) + t * ROWS_PER_TEC

    pltpu.sync_copy(asrc_hbm, asrc_v.at[pl.ds(0, N)])
    pltpu.sync_copy(adst_hbm, adst_v.at[pl.ds(0, N)])
    pltpu.sync_copy(srcr_hbm.at[pl.ds(rowbase, ROWS_PER_TEC)], src_v)
    pltpu.sync_copy(dstr_hbm.at[pl.ds(rowbase, ROWS_PER_TEC)], dst_v)

    def _zb(i, carry):
        zb_v[pl.ds(i * 16, 16)] = jnp.zeros((16,), jnp.float32)
        return carry
    lax.fori_loop(0, NSLICE // 16, _zb, 0)
    pltpu.sync_copy(zb_v, den_sp.at[pl.ds(t * NSLICE, NSLICE)])

    def _row(b, carry):
        valid = (rowbase + b) < EROWS_VALID

        def _grp(j, carry2):
            sl = pl.ds(j * 16, 16)
            si = src_v[b, sl]
            di = dst_v[b, sl]
            av = plsc.load_gather(asrc_v, [si])
            bv = plsc.load_gather(adst_v, [di])
            alpha = 1.0 / (1.0 + jnp.exp(-(av + bv)))
            ex = jnp.exp(alpha)
            ex = jnp.where(valid, ex, jnp.zeros((16,), jnp.float32))
            ex_v[b, sl] = ex
            return carry2
        lax.fori_loop(0, 8, _grp, 0)
        return carry
    lax.fori_loop(0, ROWS_PER_TEC, _row, 0)

    pltpu.sync_copy(ex_v, exr_hbm.at[pl.ds(rowbase, ROWS_PER_TEC)])

    plsc.subcore_barrier()

    def _scat(b, carry):
        pltpu.sync_copy(ex_v.at[b], den_sp.at[dst_v.at[b]], add=True)
        return carry
    lax.fori_loop(0, ROWS_PER_TEC, _scat, 0)

    plsc.subcore_barrier()
    pltpu.sync_copy(den_sp.at[pl.ds(t * NSLICE, NSLICE)],
                    den2_hbm.at[c, pl.ds(t * NSLICE, NSLICE)])


_edge_w = pl.kernel(
    _edge_w_body,
    out_type=[
        jax.ShapeDtypeStruct((ER, 128), jnp.float32),
        jax.ShapeDtypeStruct((NUM_CORES, NPAD), jnp.float32),
    ],
    mesh=_MESH,
    scratch_types=[
        pltpu.VMEM((NPAD,), jnp.float32),
        pltpu.VMEM((NPAD,), jnp.float32),
        pltpu.VMEM((ROWS_PER_TEC, 128), jnp.int32),
        pltpu.VMEM((ROWS_PER_TEC, 128), jnp.int32),
        pltpu.VMEM((ROWS_PER_TEC, 128), jnp.float32),
        pltpu.VMEM((NSLICE,), jnp.float32),
        pltpu.VMEM_SHARED((NPAD,), jnp.float32),
    ],
    compiler_params=_SC_PARAMS,
)


def _spmm_body(xflat_hbm, srcr_hbm, dstr_hbm, exr_hbm, out_hbm,
               src_v, dst_v, ex_v, idx_v, rows_v, zb_v, acc_sp, sem):
    c = lax.axis_index("c")
    t = lax.axis_index("s")
    rowbase = c * (NUM_SUBCORES * ROWS_PER_TEC) + t * ROWS_PER_TEC

    pltpu.sync_copy(srcr_hbm.at[pl.ds(rowbase, ROWS_PER_TEC)], src_v)
    pltpu.sync_copy(dstr_hbm.at[pl.ds(rowbase, ROWS_PER_TEC)], dst_v)
    pltpu.sync_copy(exr_hbm.at[pl.ds(rowbase, ROWS_PER_TEC)], ex_v)

    def _zr(r, carry):
        def _zj(j, carry2):
            zb_v[r, pl.ds(j * 16, 16)] = jnp.zeros((16,), jnp.float32)
            return carry2
        lax.fori_loop(0, FC // 16, _zj, 0)
        return carry
    lax.fori_loop(0, 64, _zr, 0)

    def _chunk(k, carry):
        def _z(i, carry2):
            pltpu.sync_copy(zb_v, acc_sp.at[pl.ds(t * NSLICE + i * 64, 64)])
            return carry2
        lax.fori_loop(0, NSLICE // 64, _z, 0)

        base = k * N

        def _bi(b, carry2):
            def _bj(j, carry3):
                sl = pl.ds(j * 16, 16)
                idx_v[b, sl] = src_v[b, sl] + base
                return carry3
            lax.fori_loop(0, 8, _bj, 0)
            return carry2
        lax.fori_loop(0, ROWS_PER_TEC, _bi, 0)

        plsc.subcore_barrier()

        def _edge_batch(b, carry2):
            pltpu.async_copy(xflat_hbm.at[idx_v.at[b]], rows_v, sem).wait()

            def _scale_row(r, carry3):
                w = plsc.load_gather(
                    ex_v, [jnp.full((16,), b, jnp.int32),
                           jnp.full((16,), r, jnp.int32)])

                def _cj(j, carry4):
                    sl = pl.ds(j * 16, 16)
                    rows_v[r, sl] = rows_v[r, sl] * w
                    return carry4
                lax.fori_loop(0, FC // 16, _cj, 0)
                return carry3
            lax.fori_loop(0, 128, _scale_row, 0)

            pltpu.sync_copy(rows_v, acc_sp.at[dst_v.at[b]], add=True)
            return carry2
        lax.fori_loop(0, ROWS_PER_TEC, _edge_batch, 0)

        plsc.subcore_barrier()
        pltpu.sync_copy(acc_sp.at[pl.ds(t * NSLICE, NSLICE)],
                        out_hbm.at[c, k, pl.ds(t * NSLICE, NSLICE)])
        return carry
    lax.fori_loop(0, NK, _chunk, 0)


_spmm = pl.kernel(
    _spmm_body,
    out_type=jax.ShapeDtypeStruct((NUM_CORES, NK, NPAD, FC), jnp.float32),
    mesh=_MESH,
    scratch_types=[
        pltpu.VMEM((ROWS_PER_TEC, 128), jnp.int32),
        pltpu.VMEM((ROWS_PER_TEC, 128), jnp.int32),
        pltpu.VMEM((ROWS_PER_TEC, 128), jnp.float32),
        pltpu.VMEM((ROWS_PER_TEC, 128), jnp.int32),
        pltpu.VMEM((128, FC), jnp.float32),
        pltpu.VMEM((64, FC), jnp.float32),
        pltpu.VMEM_SHARED((NPAD, FC), jnp.float32),
        pltpu.SemaphoreType.DMA,
    ],
    compiler_params=_SC_PARAMS,
)


def kernel(features, edge_index, W1, att_src1, att_dst1, W2):
    src = edge_index[0]
    dst = edge_index[1]
    pad = ER * 128 - E
    zpad = jnp.zeros((pad,), jnp.int32)
    srcr = jnp.concatenate([src, zpad]).reshape(ER, 128)
    dstr = jnp.concatenate([dst, zpad]).reshape(ER, 128)
    att2 = (jnp.zeros((D, 128), jnp.float32)
            .at[:, 0].set(att_src1).at[:, 1].set(att_dst1))

    xflat, a2 = _k1(features, W1, att2)
    a_src = a2[:, 0]
    a_dst = a2[:, 1]

    exr, den2 = _edge_w(a_src, a_dst, srcr, dstr)
    den3 = (den2[0, :N] + den2[1, :N]).reshape(N // RB, 1, RB)

    p1 = _spmm(xflat, srcr, dstr, exr)
    h2, x3 = _k2(p1, den3, W2, W2.T)

    p2 = _spmm(x3.reshape(NK * N, FC), srcr, dstr, exr)
    h4 = _k3(p2, den3, W1.T)
    return (h2, h4)


# speedup vs baseline: 3.5680x; 1.2864x over previous
import jax
import jax.numpy as jnp
from jax import lax
from jax.experimental import pallas as pl
from jax.experimental.pallas import tpu as pltpu
from jax.experimental.pallas import tpu_sc as plsc

N = 10000
E = 160000
D = 512
DO = 256
FC = 128
NK = D // FC
ER = 1280
EROWS_VALID = E // 128
NPAD = 10240
RB = 1000

NUM_CORES = 2
NUM_SUBCORES = 16
ROWS_PER_TEC = ER // (NUM_CORES * NUM_SUBCORES)
NSLICE = NPAD // NUM_SUBCORES

_MESH = plsc.VectorSubcoreMesh(
    core_axis_name="c", subcore_axis_name="s",
    num_cores=NUM_CORES, num_subcores=NUM_SUBCORES)

_SC_PARAMS = pltpu.CompilerParams(needs_layout_passes=False)


def _k1_body(feat_ref, w1_ref, att2_ref, xflat_ref, a2_ref):
    k = pl.program_id(1)
    xk = jnp.dot(feat_ref[...], w1_ref[...], preferred_element_type=jnp.float32)
    xflat_ref[...] = xk

    @pl.when(k == 0)
    def _():
        a2_ref[...] = jnp.zeros_like(a2_ref)

    a2_ref[...] += jnp.dot(xk, att2_ref[...], preferred_element_type=jnp.float32)


_k1 = pl.pallas_call(
    _k1_body,
    grid=(N // RB, NK),
    in_specs=[
        pl.BlockSpec((RB, D), lambda i, k: (i, 0)),
        pl.BlockSpec((D, FC), lambda i, k: (0, k)),
        pl.BlockSpec((FC, 128), lambda i, k: (k, 0)),
    ],
    out_specs=[
        pl.BlockSpec((RB, FC), lambda i, k: (k * (N // RB) + i, 0)),
        pl.BlockSpec((RB, 128), lambda i, k: (i, 0)),
    ],
    out_shape=[
        jax.ShapeDtypeStruct((NK * N, FC), jnp.float32),
        jax.ShapeDtypeStruct((N, 128), jnp.float32),
    ],
)


def _elu(x):
    return jnp.where(x > 0, x, jnp.exp(x) - 1.0)


def _k2_body(p_ref, d2_ref, w2_ref, w2t_ref, h2_ref, x3_ref):
    rd = 1.0 / (d2_ref[0, 0] + 1e-16)
    acc = jnp.zeros((RB, DO), jnp.float32)
    for k in range(NK):
        p = (p_ref[0, k] + p_ref[1, k]) * rd[:, None]
        h = _elu(p)
        acc += jnp.dot(h, w2_ref[k * FC:(k + 1) * FC, :],
                       preferred_element_type=jnp.float32)
    h2_ref[...] = acc
    for k in range(NK):
        x3_ref[k] = jnp.dot(acc, w2t_ref[:, k * FC:(k + 1) * FC],
                            preferred_element_type=jnp.float32)


_k2 = pl.pallas_call(
    _k2_body,
    grid=(N // RB,),
    in_specs=[
        pl.BlockSpec((NUM_CORES, NK, RB, FC), lambda i: (0, 0, i, 0)),
        pl.BlockSpec((1, 1, RB), lambda i: (i, 0, 0)),
        pl.BlockSpec((D, DO), lambda i: (0, 0)),
        pl.BlockSpec((DO, D), lambda i: (0, 0)),
    ],
    out_specs=[
        pl.BlockSpec((RB, DO), lambda i: (i, 0)),
        pl.BlockSpec((NK, RB, FC), lambda i: (0, i, 0)),
    ],
    out_shape=[
        jax.ShapeDtypeStruct((N, DO), jnp.float32),
        jax.ShapeDtypeStruct((NK, N, FC), jnp.float32),
    ],
)


def _k3_body(p_ref, d2_ref, w1t_ref, h4_ref):
    rd = 1.0 / (d2_ref[0, 0] + 1e-16)
    acc = jnp.zeros((RB, D), jnp.float32)
    for k in range(NK):
        p = (p_ref[0, k] + p_ref[1, k]) * rd[:, None]
        h = _elu(p)
        acc += jnp.dot(h, w1t_ref[k * FC:(k + 1) * FC, :],
                       preferred_element_type=jnp.float32)
    h4_ref[...] = acc


_k3 = pl.pallas_call(
    _k3_body,
    grid=(N // RB,),
    in_specs=[
        pl.BlockSpec((NUM_CORES, NK, RB, FC), lambda i: (0, 0, i, 0)),
        pl.BlockSpec((1, 1, RB), lambda i: (i, 0, 0)),
        pl.BlockSpec((D, D), lambda i: (0, 0)),
    ],
    out_specs=pl.BlockSpec((RB, D), lambda i: (i, 0)),
    out_shape=jax.ShapeDtypeStruct((N, D), jnp.float32),
)


def _edge_w_body(asrc_hbm, adst_hbm, srcr_hbm, dstr_hbm, exr_hbm, den2_hbm,
                 asrc_v, adst_v, src_v, dst_v, ex_v, zb_v, den_sp):
    c = lax.axis_index("c")
    t = lax.axis_index("s")
    rowbase = c * (NUM_SUBCORES * ROWS_PER_TEC) + t * ROWS_PER_TEC

    pltpu.sync_copy(asrc_hbm, asrc_v.at[pl.ds(0, N)])
    pltpu.sync_copy(adst_hbm, adst_v.at[pl.ds(0, N)])
    pltpu.sync_copy(srcr_hbm.at[pl.ds(rowbase, ROWS_PER_TEC)], src_v)
    pltpu.sync_copy(dstr_hbm.at[pl.ds(rowbase, ROWS_PER_TEC)], dst_v)

    def _zb(i, carry):
        zb_v[pl.ds(i * 16, 16)] = jnp.zeros((16,), jnp.float32)
        return carry
    lax.fori_loop(0, NSLICE // 16, _zb, 0)
    pltpu.sync_copy(zb_v, den_sp.at[pl.ds(t * NSLICE, NSLICE)])

    def _row(b, carry):
        valid = (rowbase + b) < EROWS_VALID

        def _grp(j, carry2):
            sl = pl.ds(j * 16, 16)
            si = src_v[b, sl]
            di = dst_v[b, sl]
            av = plsc.load_gather(asrc_v, [si])
            bv = plsc.load_gather(adst_v, [di])
            alpha = 1.0 / (1.0 + jnp.exp(-(av + bv)))
            ex = jnp.exp(alpha)
            ex = jnp.where(valid, ex, jnp.zeros((16,), jnp.float32))
            ex_v[b, sl] = ex
            return carry2
        lax.fori_loop(0, 8, _grp, 0)
        return carry
    lax.fori_loop(0, ROWS_PER_TEC, _row, 0)

    pltpu.sync_copy(ex_v, exr_hbm.at[pl.ds(rowbase, ROWS_PER_TEC)])

    plsc.subcore_barrier()

    def _scat(b, carry):
        pltpu.sync_copy(ex_v.at[b], den_sp.at[dst_v.at[b]], add=True)
        return carry
    lax.fori_loop(0, ROWS_PER_TEC, _scat, 0)

    plsc.subcore_barrier()
    pltpu.sync_copy(den_sp.at[pl.ds(t * NSLICE, NSLICE)],
                    den2_hbm.at[c, pl.ds(t * NSLICE, NSLICE)])


_edge_w = pl.kernel(
    _edge_w_body,
    out_type=[
        jax.ShapeDtypeStruct((ER, 128), jnp.float32),
        jax.ShapeDtypeStruct((NUM_CORES, NPAD), jnp.float32),
    ],
    mesh=_MESH,
    scratch_types=[
        pltpu.VMEM((NPAD,), jnp.float32),
        pltpu.VMEM((NPAD,), jnp.float32),
        pltpu.VMEM((ROWS_PER_TEC, 128), jnp.int32),
        pltpu.VMEM((ROWS_PER_TEC, 128), jnp.int32),
        pltpu.VMEM((ROWS_PER_TEC, 128), jnp.float32),
        pltpu.VMEM((NSLICE,), jnp.float32),
        pltpu.VMEM_SHARED((NPAD,), jnp.float32),
    ],
    compiler_params=_SC_PARAMS,
)


def _spmm_body(xflat_hbm, srcr_hbm, dstr_hbm, exr_hbm, out_hbm,
               src_v, dst_v, ex_v, rows0_v, rows1_v, acc_sp,
               gsem0, gsem1, ssem0, ssem1):
    c = lax.axis_index("c")
    t = lax.axis_index("s")
    rowbase = c * (NUM_SUBCORES * ROWS_PER_TEC) + t * ROWS_PER_TEC
    rows = (rows0_v, rows1_v)
    gsem = (gsem0, gsem1)
    ssem = (ssem0, ssem1)

    pltpu.sync_copy(srcr_hbm.at[pl.ds(rowbase, ROWS_PER_TEC)], src_v)
    pltpu.sync_copy(dstr_hbm.at[pl.ds(rowbase, ROWS_PER_TEC)], dst_v)
    pltpu.sync_copy(exr_hbm.at[pl.ds(rowbase, ROWS_PER_TEC)], ex_v)

    def _scale(buf, b):
        def _scale_row(r, carry):
            w = plsc.load_gather(
                ex_v, [jnp.full((16,), b, jnp.int32),
                       jnp.full((16,), r, jnp.int32)])

            def _cj(j, carry2):
                sl = pl.ds(j * 16, 16)
                buf[r, sl] = buf[r, sl] * w
                return carry2
            lax.fori_loop(0, FC // 16, _cj, 0)
            return carry
        lax.fori_loop(0, 128, _scale_row, 0)

    def _chunk(k, carry):
        @pl.when(k > 0)
        def _():
            def _bi(b, carry2):
                def _bj(j, carry3):
                    sl = pl.ds(j * 16, 16)
                    src_v[b, sl] = src_v[b, sl] + N
                    return carry3
                lax.fori_loop(0, 8, _bj, 0)
                return carry2
            lax.fori_loop(0, ROWS_PER_TEC, _bi, 0)

        def _zr(r, carry2):
            def _zj(j, carry3):
                rows0_v[r, pl.ds(j * 16, 16)] = jnp.zeros((16,), jnp.float32)
                return carry3
            lax.fori_loop(0, FC // 16, _zj, 0)
            return carry2
        lax.fori_loop(0, 128, _zr, 0)

        def _z(i, carry2):
            pltpu.sync_copy(rows0_v,
                            acc_sp.at[pl.ds(t * NSLICE + i * 128, 128)])
            return carry2
        lax.fori_loop(0, NSLICE // 128, _z, 0)

        plsc.subcore_barrier()

        pltpu.async_copy(xflat_hbm.at[src_v.at[0]], rows0_v, gsem0)
        pltpu.async_copy(xflat_hbm.at[src_v.at[1]], rows1_v, gsem1)

        def _edge_pair(b2, carry2):
            for ph in range(2):
                b = b2 * 2 + ph
                pltpu.make_async_copy(
                    xflat_hbm.at[src_v.at[b]], rows[ph], gsem[ph]).wait()
                _scale(rows[ph], b)
                pltpu.async_copy(
                    rows[ph], acc_sp.at[dst_v.at[b]], ssem[ph], add=True)
                pltpu.make_async_copy(
                    rows[ph], acc_sp.at[dst_v.at[b]], ssem[ph]).wait()
                pltpu.async_copy(
                    xflat_hbm.at[src_v.at[b + 2]], rows[ph], gsem[ph])
            return carry2
        lax.fori_loop(0, ROWS_PER_TEC // 2 - 1, _edge_pair, 0)

        for ph in range(2):
            b = ROWS_PER_TEC - 2 + ph
            pltpu.make_async_copy(
                xflat_hbm.at[src_v.at[b]], rows[ph], gsem[ph]).wait()
            _scale(rows[ph], b)
            pltpu.sync_copy(rows[ph], acc_sp.at[dst_v.at[b]], add=True)

        plsc.subcore_barrier()
        pltpu.sync_copy(acc_sp.at[pl.ds(t * NSLICE, NSLICE)],
                        out_hbm.at[c, k, pl.ds(t * NSLICE, NSLICE)])
        return carry
    lax.fori_loop(0, NK, _chunk, 0)


_spmm = pl.kernel(
    _spmm_body,
    out_type=jax.ShapeDtypeStruct((NUM_CORES, NK, NPAD, FC), jnp.float32),
    mesh=_MESH,
    scratch_types=[
        pltpu.VMEM((ROWS_PER_TEC, 128), jnp.int32),
        pltpu.VMEM((ROWS_PER_TEC, 128), jnp.int32),
        pltpu.VMEM((ROWS_PER_TEC, 128), jnp.float32),
        pltpu.VMEM((128, FC), jnp.float32),
        pltpu.VMEM((128, FC), jnp.float32),
        pltpu.VMEM_SHARED((NPAD, FC), jnp.float32),
        pltpu.SemaphoreType.DMA,
        pltpu.SemaphoreType.DMA,
        pltpu.SemaphoreType.DMA,
        pltpu.SemaphoreType.DMA,
    ],
    compiler_params=_SC_PARAMS,
)


def kernel(features, edge_index, W1, att_src1, att_dst1, W2):
    src = edge_index[0]
    dst = edge_index[1]
    pad = ER * 128 - E
    zpad = jnp.zeros((pad,), jnp.int32)
    srcr = jnp.concatenate([src, zpad]).reshape(ER, 128)
    dstr = jnp.concatenate([dst, zpad]).reshape(ER, 128)
    att2 = (jnp.zeros((D, 128), jnp.float32)
            .at[:, 0].set(att_src1).at[:, 1].set(att_dst1))

    xflat, a2 = _k1(features, W1, att2)
    a_src = a2[:, 0]
    a_dst = a2[:, 1]

    exr, den2 = _edge_w(a_src, a_dst, srcr, dstr)
    den3 = (den2[0, :N] + den2[1, :N]).reshape(N // RB, 1, RB)

    p1 = _spmm(xflat, srcr, dstr, exr)
    h2, x3 = _k2(p1, den3, W2, W2.T)

    p2 = _spmm(x3.reshape(NK * N, FC), srcr, dstr, exr)
    h4 = _k3(p2, den3, W1.T)
    return (h2, h4)


# speedup vs baseline: 3.6000x; 1.0090x over previous
import jax
import jax.numpy as jnp
from jax import lax
from jax.experimental import pallas as pl
from jax.experimental.pallas import tpu as pltpu
from jax.experimental.pallas import tpu_sc as plsc

N = 10000
E = 160000
D = 512
DO = 256
FC = 128
NK = D // FC
ER = 1280
EROWS_VALID = E // 128
NPAD = 10240
RB = 1000

NUM_CORES = 2
NUM_SUBCORES = 16
ROWS_PER_TEC = ER // (NUM_CORES * NUM_SUBCORES)
NSLICE = NPAD // NUM_SUBCORES

_MESH = plsc.VectorSubcoreMesh(
    core_axis_name="c", subcore_axis_name="s",
    num_cores=NUM_CORES, num_subcores=NUM_SUBCORES)

_SC_PARAMS = pltpu.CompilerParams(needs_layout_passes=False)


def _k1_body(feat_ref, w1_ref, att2_ref, xflat_ref, a2_ref):
    k = pl.program_id(1)
    xk = jnp.dot(feat_ref[...], w1_ref[...], preferred_element_type=jnp.float32)
    xflat_ref[...] = xk

    @pl.when(k == 0)
    def _():
        a2_ref[...] = jnp.zeros_like(a2_ref)

    a2_ref[...] += jnp.dot(xk, att2_ref[...], preferred_element_type=jnp.float32)


_k1 = pl.pallas_call(
    _k1_body,
    grid=(N // RB, NK),
    in_specs=[
        pl.BlockSpec((RB, D), lambda i, k: (i, 0)),
        pl.BlockSpec((D, FC), lambda i, k: (0, k)),
        pl.BlockSpec((FC, 128), lambda i, k: (k, 0)),
    ],
    out_specs=[
        pl.BlockSpec((RB, FC), lambda i, k: (k * (N // RB) + i, 0)),
        pl.BlockSpec((RB, 128), lambda i, k: (i, 0)),
    ],
    out_shape=[
        jax.ShapeDtypeStruct((NK * N, FC), jnp.float32),
        jax.ShapeDtypeStruct((N, 128), jnp.float32),
    ],
)


def _elu(x):
    return jnp.where(x > 0, x, jnp.exp(x) - 1.0)


def _k2_body(p_ref, d2_ref, w2_ref, w2t_ref, h2_ref, x3_ref):
    rd = 1.0 / (d2_ref[0, 0] + 1e-16)
    acc = jnp.zeros((RB, DO), jnp.float32)
    for k in range(NK):
        p = (p_ref[0, k] + p_ref[1, k]) * rd[:, None]
        h = _elu(p)
        acc += jnp.dot(h, w2_ref[k * FC:(k + 1) * FC, :],
                       preferred_element_type=jnp.float32)
    h2_ref[...] = acc
    for k in range(NK):
        x3_ref[k] = jnp.dot(acc, w2t_ref[:, k * FC:(k + 1) * FC],
                            preferred_element_type=jnp.float32)


_k2 = pl.pallas_call(
    _k2_body,
    grid=(N // RB,),
    in_specs=[
        pl.BlockSpec((NUM_CORES, NK, RB, FC), lambda i: (0, 0, i, 0)),
        pl.BlockSpec((1, 1, RB), lambda i: (i, 0, 0)),
        pl.BlockSpec((D, DO), lambda i: (0, 0)),
        pl.BlockSpec((DO, D), lambda i: (0, 0)),
    ],
    out_specs=[
        pl.BlockSpec((RB, DO), lambda i: (i, 0)),
        pl.BlockSpec((NK, RB, FC), lambda i: (0, i, 0)),
    ],
    out_shape=[
        jax.ShapeDtypeStruct((N, DO), jnp.float32),
        jax.ShapeDtypeStruct((NK, N, FC), jnp.float32),
    ],
)


def _k3_body(p_ref, d2_ref, w1t_ref, h4_ref):
    rd = 1.0 / (d2_ref[0, 0] + 1e-16)
    acc = jnp.zeros((RB, D), jnp.float32)
    for k in range(NK):
        p = (p_ref[0, k] + p_ref[1, k]) * rd[:, None]
        h = _elu(p)
        acc += jnp.dot(h, w1t_ref[k * FC:(k + 1) * FC, :],
                       preferred_element_type=jnp.float32)
    h4_ref[...] = acc


_k3 = pl.pallas_call(
    _k3_body,
    grid=(N // RB,),
    in_specs=[
        pl.BlockSpec((NUM_CORES, NK, RB, FC), lambda i: (0, 0, i, 0)),
        pl.BlockSpec((1, 1, RB), lambda i: (i, 0, 0)),
        pl.BlockSpec((D, D), lambda i: (0, 0)),
    ],
    out_specs=pl.BlockSpec((RB, D), lambda i: (i, 0)),
    out_shape=jax.ShapeDtypeStruct((N, D), jnp.float32),
)


def _edge_w_body(asrc_hbm, adst_hbm, srcr_hbm, dstr_hbm, exr_hbm, den2_hbm,
                 asrc_v, adst_v, src_v, dst_v, ex_v, zb_v, den_sp):
    c = lax.axis_index("c")
    t = lax.axis_index("s")
    rowbase = c * (NUM_SUBCORES * ROWS_PER_TEC) + t * ROWS_PER_TEC

    pltpu.sync_copy(asrc_hbm, asrc_v.at[pl.ds(0, N)])
    pltpu.sync_copy(adst_hbm, adst_v.at[pl.ds(0, N)])
    pltpu.sync_copy(srcr_hbm.at[pl.ds(rowbase, ROWS_PER_TEC)], src_v)
    pltpu.sync_copy(dstr_hbm.at[pl.ds(rowbase, ROWS_PER_TEC)], dst_v)

    def _zb(i, carry):
        zb_v[pl.ds(i * 16, 16)] = jnp.zeros((16,), jnp.float32)
        return carry
    lax.fori_loop(0, NSLICE // 16, _zb, 0)
    pltpu.sync_copy(zb_v, den_sp.at[pl.ds(t * NSLICE, NSLICE)])

    def _row(b, carry):
        valid = (rowbase + b) < EROWS_VALID

        def _grp(j, carry2):
            sl = pl.ds(j * 16, 16)
            si = src_v[b, sl]
            di = dst_v[b, sl]
            av = plsc.load_gather(asrc_v, [si])
            bv = plsc.load_gather(adst_v, [di])
            alpha = 1.0 / (1.0 + jnp.exp(-(av + bv)))
            ex = jnp.exp(alpha)
            ex = jnp.where(valid, ex, jnp.zeros((16,), jnp.float32))
            ex_v[b, sl] = ex
            return carry2
        lax.fori_loop(0, 8, _grp, 0)
        return carry
    lax.fori_loop(0, ROWS_PER_TEC, _row, 0)

    pltpu.sync_copy(ex_v, exr_hbm.at[pl.ds(rowbase, ROWS_PER_TEC)])

    plsc.subcore_barrier()

    def _scat(b, carry):
        pltpu.sync_copy(ex_v.at[b], den_sp.at[dst_v.at[b]], add=True)
        return carry
    lax.fori_loop(0, ROWS_PER_TEC, _scat, 0)

    plsc.subcore_barrier()
    pltpu.sync_copy(den_sp.at[pl.ds(t * NSLICE, NSLICE)],
                    den2_hbm.at[c, pl.ds(t * NSLICE, NSLICE)])


_edge_w = pl.kernel(
    _edge_w_body,
    out_type=[
        jax.ShapeDtypeStruct((ER, 128), jnp.float32),
        jax.ShapeDtypeStruct((NUM_CORES, NPAD), jnp.float32),
    ],
    mesh=_MESH,
    scratch_types=[
        pltpu.VMEM((NPAD,), jnp.float32),
        pltpu.VMEM((NPAD,), jnp.float32),
        pltpu.VMEM((ROWS_PER_TEC, 128), jnp.int32),
        pltpu.VMEM((ROWS_PER_TEC, 128), jnp.int32),
        pltpu.VMEM((ROWS_PER_TEC, 128), jnp.float32),
        pltpu.VMEM((NSLICE,), jnp.float32),
        pltpu.VMEM_SHARED((NPAD,), jnp.float32),
    ],
    compiler_params=_SC_PARAMS,
)


def _spmm_body(xflat_hbm, srcr_hbm, dstr_hbm, exr_hbm, out_hbm,
               src_v, dst_v, ex_v, rows0_v, rows1_v, acc_sp,
               gsem0, gsem1, ssem0, ssem1):
    c = lax.axis_index("c")
    t = lax.axis_index("s")
    rowbase = c * (NUM_SUBCORES * ROWS_PER_TEC) + t * ROWS_PER_TEC
    rows = (rows0_v, rows1_v)
    gsem = (gsem0, gsem1)
    ssem = (ssem0, ssem1)

    pltpu.sync_copy(srcr_hbm.at[pl.ds(rowbase, ROWS_PER_TEC)], src_v)
    pltpu.sync_copy(dstr_hbm.at[pl.ds(rowbase, ROWS_PER_TEC)], dst_v)
    pltpu.sync_copy(exr_hbm.at[pl.ds(rowbase, ROWS_PER_TEC)], ex_v)

    def _scale(buf, b):
        @plsc.parallel_loop(0, 128, unroll=2)
        def _scale_row(r):
            w = plsc.load_gather(
                ex_v, [jnp.full((16,), b, jnp.int32),
                       jnp.full((16,), r, jnp.int32)])
            for j in range(FC // 16):
                sl = pl.ds(j * 16, 16)
                buf[r, sl] = buf[r, sl] * w

    def _chunk(k, carry):
        @pl.when(k > 0)
        def _():
            @plsc.parallel_loop(0, ROWS_PER_TEC)
            def _bi(b):
                for j in range(8):
                    sl = pl.ds(j * 16, 16)
                    src_v[b, sl] = src_v[b, sl] + N

        @plsc.parallel_loop(0, 128, unroll=2)
        def _zr(r):
            for j in range(FC // 16):
                rows0_v[r, pl.ds(j * 16, 16)] = jnp.zeros((16,), jnp.float32)

        def _z(i, carry2):
            pltpu.sync_copy(rows0_v,
                            acc_sp.at[pl.ds(t * NSLICE + i * 128, 128)])
            return carry2
        lax.fori_loop(0, NSLICE // 128, _z, 0)

        plsc.subcore_barrier()

        pltpu.async_copy(xflat_hbm.at[src_v.at[0]], rows0_v, gsem0)
        pltpu.async_copy(xflat_hbm.at[src_v.at[1]], rows1_v, gsem1)

        def _edge_pair(b2, carry2):
            for ph in range(2):
                b = b2 * 2 + ph
                pltpu.make_async_copy(
                    xflat_hbm.at[src_v.at[b]], rows[ph], gsem[ph]).wait()
                _scale(rows[ph], b)
                pltpu.async_copy(
                    rows[ph], acc_sp.at[dst_v.at[b]], ssem[ph], add=True)
                pltpu.make_async_copy(
                    rows[ph], acc_sp.at[dst_v.at[b]], ssem[ph]).wait()
                pltpu.async_copy(
                    xflat_hbm.at[src_v.at[b + 2]], rows[ph], gsem[ph])
            return carry2
        lax.fori_loop(0, ROWS_PER_TEC // 2 - 1, _edge_pair, 0)

        for ph in range(2):
            b = ROWS_PER_TEC - 2 + ph
            pltpu.make_async_copy(
                xflat_hbm.at[src_v.at[b]], rows[ph], gsem[ph]).wait()
            _scale(rows[ph], b)
            pltpu.sync_copy(rows[ph], acc_sp.at[dst_v.at[b]], add=True)

        plsc.subcore_barrier()
        pltpu.sync_copy(acc_sp.at[pl.ds(t * NSLICE, NSLICE)],
                        out_hbm.at[c, k, pl.ds(t * NSLICE, NSLICE)])
        return carry
    lax.fori_loop(0, NK, _chunk, 0)


_spmm = pl.kernel(
    _spmm_body,
    out_type=jax.ShapeDtypeStruct((NUM_CORES, NK, NPAD, FC), jnp.float32),
    mesh=_MESH,
    scratch_types=[
        pltpu.VMEM((ROWS_PER_TEC, 128), jnp.int32),
        pltpu.VMEM((ROWS_PER_TEC, 128), jnp.int32),
        pltpu.VMEM((ROWS_PER_TEC, 128), jnp.float32),
        pltpu.VMEM((128, FC), jnp.float32),
        pltpu.VMEM((128, FC), jnp.float32),
        pltpu.VMEM_SHARED((NPAD, FC), jnp.float32),
        pltpu.SemaphoreType.DMA,
        pltpu.SemaphoreType.DMA,
        pltpu.SemaphoreType.DMA,
        pltpu.SemaphoreType.DMA,
    ],
    compiler_params=_SC_PARAMS,
)


def kernel(features, edge_index, W1, att_src1, att_dst1, W2):
    src = edge_index[0]
    dst = edge_index[1]
    pad = ER * 128 - E
    zpad = jnp.zeros((pad,), jnp.int32)
    srcr = jnp.concatenate([src, zpad]).reshape(ER, 128)
    dstr = jnp.concatenate([dst, zpad]).reshape(ER, 128)
    att2 = (jnp.zeros((D, 128), jnp.float32)
            .at[:, 0].set(att_src1).at[:, 1].set(att_dst1))

    xflat, a2 = _k1(features, W1, att2)
    a_src = a2[:, 0]
    a_dst = a2[:, 1]

    exr, den2 = _edge_w(a_src, a_dst, srcr, dstr)
    den3 = (den2[0, :N] + den2[1, :N]).reshape(N // RB, 1, RB)

    p1 = _spmm(xflat, srcr, dstr, exr)
    h2, x3 = _k2(p1, den3, W2, W2.T)

    p2 = _spmm(x3.reshape(NK * N, FC), srcr, dstr, exr)
    h4 = _k3(p2, den3, W1.T)
    return (h2, h4)


# speedup vs baseline: 3.6319x; 1.0088x over previous
import jax
import jax.numpy as jnp
from jax import lax
from jax.experimental import pallas as pl
from jax.experimental.pallas import tpu as pltpu
from jax.experimental.pallas import tpu_sc as plsc

N = 10000
E = 160000
D = 512
DO = 256
FC = 128
NK = D // FC
ER = 1280
EROWS_VALID = E // 128
NPAD = 10240
RB = 1000

NUM_CORES = 2
NUM_SUBCORES = 16
ROWS_PER_TEC = ER // (NUM_CORES * NUM_SUBCORES)
B_PER_TEC = 2 * ROWS_PER_TEC
NSLICE = NPAD // NUM_SUBCORES

_MESH = plsc.VectorSubcoreMesh(
    core_axis_name="c", subcore_axis_name="s",
    num_cores=NUM_CORES, num_subcores=NUM_SUBCORES)

_SC_PARAMS = pltpu.CompilerParams(needs_layout_passes=False)


def _k1_body(feat_ref, w1_ref, att2_ref, xflat_ref, a2_ref):
    k = pl.program_id(1)
    xk = jnp.dot(feat_ref[...], w1_ref[...], preferred_element_type=jnp.float32)
    xflat_ref[...] = xk

    @pl.when(k == 0)
    def _():
        a2_ref[...] = jnp.zeros_like(a2_ref)

    a2_ref[...] += jnp.dot(xk, att2_ref[...], preferred_element_type=jnp.float32)


_k1 = pl.pallas_call(
    _k1_body,
    grid=(N // RB, NK),
    in_specs=[
        pl.BlockSpec((RB, D), lambda i, k: (i, 0)),
        pl.BlockSpec((D, FC), lambda i, k: (0, k)),
        pl.BlockSpec((FC, 128), lambda i, k: (k, 0)),
    ],
    out_specs=[
        pl.BlockSpec((RB, FC), lambda i, k: (k * (N // RB) + i, 0)),
        pl.BlockSpec((RB, 128), lambda i, k: (i, 0)),
    ],
    out_shape=[
        jax.ShapeDtypeStruct((NK * N, FC), jnp.float32),
        jax.ShapeDtypeStruct((N, 128), jnp.float32),
    ],
)


def _elu(x):
    return jnp.where(x > 0, x, jnp.exp(x) - 1.0)


def _k2_body(p_ref, d2_ref, w2_ref, w2t_ref, h2_ref, x3_ref):
    rd = 1.0 / (d2_ref[0, 0] + 1e-16)
    acc = jnp.zeros((RB, DO), jnp.float32)
    for k in range(NK):
        p = (p_ref[0, k] + p_ref[1, k]) * rd[:, None]
        h = _elu(p)
        acc += jnp.dot(h, w2_ref[k * FC:(k + 1) * FC, :],
                       preferred_element_type=jnp.float32)
    h2_ref[...] = acc
    for k in range(NK):
        x3_ref[k] = jnp.dot(acc, w2t_ref[:, k * FC:(k + 1) * FC],
                            preferred_element_type=jnp.float32)


_k2 = pl.pallas_call(
    _k2_body,
    grid=(N // RB,),
    in_specs=[
        pl.BlockSpec((NUM_CORES, NK, RB, FC), lambda i: (0, 0, i, 0)),
        pl.BlockSpec((1, 1, RB), lambda i: (i, 0, 0)),
        pl.BlockSpec((D, DO), lambda i: (0, 0)),
        pl.BlockSpec((DO, D), lambda i: (0, 0)),
    ],
    out_specs=[
        pl.BlockSpec((RB, DO), lambda i: (i, 0)),
        pl.BlockSpec((NK, RB, FC), lambda i: (0, i, 0)),
    ],
    out_shape=[
        jax.ShapeDtypeStruct((N, DO), jnp.float32),
        jax.ShapeDtypeStruct((NK, N, FC), jnp.float32),
    ],
)


def _k3_body(p_ref, d2_ref, w1t_ref, h4_ref):
    rd = 1.0 / (d2_ref[0, 0] + 1e-16)
    acc = jnp.zeros((RB, D), jnp.float32)
    for k in range(NK):
        p = (p_ref[0, k] + p_ref[1, k]) * rd[:, None]
        h = _elu(p)
        acc += jnp.dot(h, w1t_ref[k * FC:(k + 1) * FC, :],
                       preferred_element_type=jnp.float32)
    h4_ref[...] = acc


_k3 = pl.pallas_call(
    _k3_body,
    grid=(N // RB,),
    in_specs=[
        pl.BlockSpec((NUM_CORES, NK, RB, FC), lambda i: (0, 0, i, 0)),
        pl.BlockSpec((1, 1, RB), lambda i: (i, 0, 0)),
        pl.BlockSpec((D, D), lambda i: (0, 0)),
    ],
    out_specs=pl.BlockSpec((RB, D), lambda i: (i, 0)),
    out_shape=jax.ShapeDtypeStruct((N, D), jnp.float32),
)


def _edge_w_body(asrc_hbm, adst_hbm, srcr_hbm, dstr_hbm, exr_hbm, den2_hbm,
                 asrc_v, adst_v, src_v, dst_v, ex_v, zb_v, den_sp):
    c = lax.axis_index("c")
    t = lax.axis_index("s")
    rowbase = c * (NUM_SUBCORES * ROWS_PER_TEC) + t * ROWS_PER_TEC

    pltpu.sync_copy(asrc_hbm, asrc_v.at[pl.ds(0, N)])
    pltpu.sync_copy(adst_hbm, adst_v.at[pl.ds(0, N)])
    pltpu.sync_copy(srcr_hbm.at[pl.ds(rowbase, ROWS_PER_TEC)], src_v)
    pltpu.sync_copy(dstr_hbm.at[pl.ds(rowbase, ROWS_PER_TEC)], dst_v)

    def _zb(i, carry):
        zb_v[pl.ds(i * 16, 16)] = jnp.zeros((16,), jnp.float32)
        return carry
    lax.fori_loop(0, NSLICE // 16, _zb, 0)
    pltpu.sync_copy(zb_v, den_sp.at[pl.ds(t * NSLICE, NSLICE)])

    def _row(b, carry):
        valid = (rowbase + b) < EROWS_VALID

        def _grp(j, carry2):
            sl = pl.ds(j * 16, 16)
            si = src_v[b, sl]
            di = dst_v[b, sl]
            av = plsc.load_gather(asrc_v, [si])
            bv = plsc.load_gather(adst_v, [di])
            alpha = 1.0 / (1.0 + jnp.exp(-(av + bv)))
            ex = jnp.exp(alpha)
            ex = jnp.where(valid, ex, jnp.zeros((16,), jnp.float32))
            ex_v[b, sl] = ex
            return carry2
        lax.fori_loop(0, 8, _grp, 0)
        return carry
    lax.fori_loop(0, ROWS_PER_TEC, _row, 0)

    pltpu.sync_copy(ex_v, exr_hbm.at[pl.ds(rowbase, ROWS_PER_TEC)])

    plsc.subcore_barrier()

    def _scat(b, carry):
        pltpu.sync_copy(ex_v.at[b], den_sp.at[dst_v.at[b]], add=True)
        return carry
    lax.fori_loop(0, ROWS_PER_TEC, _scat, 0)

    plsc.subcore_barrier()
    pltpu.sync_copy(den_sp.at[pl.ds(t * NSLICE, NSLICE)],
                    den2_hbm.at[c, pl.ds(t * NSLICE, NSLICE)])


_edge_w = pl.kernel(
    _edge_w_body,
    out_type=[
        jax.ShapeDtypeStruct((ER, 128), jnp.float32),
        jax.ShapeDtypeStruct((NUM_CORES, NPAD), jnp.float32),
    ],
    mesh=_MESH,
    scratch_types=[
        pltpu.VMEM((NPAD,), jnp.float32),
        pltpu.VMEM((NPAD,), jnp.float32),
        pltpu.VMEM((ROWS_PER_TEC, 128), jnp.int32),
        pltpu.VMEM((ROWS_PER_TEC, 128), jnp.int32),
        pltpu.VMEM((ROWS_PER_TEC, 128), jnp.float32),
        pltpu.VMEM((NSLICE,), jnp.float32),
        pltpu.VMEM_SHARED((NPAD,), jnp.float32),
    ],
    compiler_params=_SC_PARAMS,
)


def _spmm_body(xflat_hbm, srcr_hbm, dstr_hbm, exr_hbm, out_hbm,
               src_v, dst_v, ex_v, g0_v, g1_v, s0_v, s1_v, acc_sp,
               gsem0, gsem1, ssem0, ssem1):
    c = lax.axis_index("c")
    t = lax.axis_index("s")
    rowbase = c * (NUM_SUBCORES * ROWS_PER_TEC) + t * ROWS_PER_TEC
    g = (g0_v, g1_v)
    s = (s0_v, s1_v)
    gsem = (gsem0, gsem1)
    ssem = (ssem0, ssem1)

    pltpu.sync_copy(srcr_hbm.at[pl.ds(rowbase, ROWS_PER_TEC)], src_v)
    pltpu.sync_copy(dstr_hbm.at[pl.ds(rowbase, ROWS_PER_TEC)], dst_v)
    pltpu.sync_copy(exr_hbm.at[pl.ds(rowbase, ROWS_PER_TEC)], ex_v)

    def _scale(b2, ph, gb, sb):
        @plsc.parallel_loop(0, 64, unroll=2)
        def _row(r):
            w = plsc.load_gather(
                ex_v, [jnp.full((16,), b2, jnp.int32),
                       jnp.full((16,), ph * 64 + r, jnp.int32)])
            for j in range(FC // 16):
                sl = pl.ds(j * 16, 16)
                sb[r, sl] = gb[r, sl] * w

    def _chunk(k, carry):
        @pl.when(k > 0)
        def _():
            @plsc.parallel_loop(0, ROWS_PER_TEC)
            def _bi(b):
                for j in range(8):
                    sl = pl.ds(j * 16, 16)
                    src_v[b, sl] = src_v[b, sl] + N

        @plsc.parallel_loop(0, 64, unroll=2)
        def _zr(r):
            for j in range(FC // 16):
                s0_v[r, pl.ds(j * 16, 16)] = jnp.zeros((16,), jnp.float32)

        def _z(i, carry2):
            pltpu.sync_copy(s0_v, acc_sp.at[pl.ds(t * NSLICE + i * 64, 64)])
            return carry2
        lax.fori_loop(0, NSLICE // 64, _z, 0)

        plsc.subcore_barrier()

        pltpu.async_copy(xflat_hbm.at[src_v.at[0, pl.ds(0, 64)]], g0_v, gsem0)
        pltpu.async_copy(xflat_hbm.at[src_v.at[0, pl.ds(64, 64)]], g1_v, gsem1)

        def _pair(b2, carry2):
            for ph in range(2):
                half = pl.ds(ph * 64, 64)
                pltpu.make_async_copy(
                    xflat_hbm.at[src_v.at[b2, half]], g[ph], gsem[ph]).wait()

                @pl.when(b2 > 0)
                def _():
                    pltpu.make_async_copy(
                        s[ph], acc_sp.at[dst_v.at[b2 - 1, half]],
                        ssem[ph]).wait()

                _scale(b2, ph, g[ph], s[ph])

                @pl.when(b2 < ROWS_PER_TEC - 1)
                def _():
                    pltpu.async_copy(
                        xflat_hbm.at[src_v.at[b2 + 1, half]], g[ph], gsem[ph])

                pltpu.async_copy(
                    s[ph], acc_sp.at[dst_v.at[b2, half]], ssem[ph], add=True)
            return carry2
        lax.fori_loop(0, ROWS_PER_TEC, _pair, 0)

        for ph in range(2):
            half = pl.ds(ph * 64, 64)
            pltpu.make_async_copy(
                s[ph], acc_sp.at[dst_v.at[ROWS_PER_TEC - 1, half]],
                ssem[ph]).wait()

        plsc.subcore_barrier()
        pltpu.sync_copy(acc_sp.at[pl.ds(t * NSLICE, NSLICE)],
                        out_hbm.at[c, k, pl.ds(t * NSLICE, NSLICE)])
        return carry
    lax.fori_loop(0, NK, _chunk, 0)


_spmm = pl.kernel(
    _spmm_body,
    out_type=jax.ShapeDtypeStruct((NUM_CORES, NK, NPAD, FC), jnp.float32),
    mesh=_MESH,
    scratch_types=[
        pltpu.VMEM((ROWS_PER_TEC, 128), jnp.int32),
        pltpu.VMEM((ROWS_PER_TEC, 128), jnp.int32),
        pltpu.VMEM((ROWS_PER_TEC, 128), jnp.float32),
        pltpu.VMEM((64, FC), jnp.float32),
        pltpu.VMEM((64, FC), jnp.float32),
        pltpu.VMEM((64, FC), jnp.float32),
        pltpu.VMEM((64, FC), jnp.float32),
        pltpu.VMEM_SHARED((NPAD, FC), jnp.float32),
        pltpu.SemaphoreType.DMA,
        pltpu.SemaphoreType.DMA,
        pltpu.SemaphoreType.DMA,
        pltpu.SemaphoreType.DMA,
    ],
    compiler_params=_SC_PARAMS,
)


def kernel(features, edge_index, W1, att_src1, att_dst1, W2):
    src = edge_index[0]
    dst = edge_index[1]
    pad = ER * 128 - E
    zpad = jnp.zeros((pad,), jnp.int32)
    srcr = jnp.concatenate([src, zpad]).reshape(ER, 128)
    dstr = jnp.concatenate([dst, zpad]).reshape(ER, 128)
    att2 = (jnp.zeros((D, 128), jnp.float32)
            .at[:, 0].set(att_src1).at[:, 1].set(att_dst1))

    xflat, a2 = _k1(features, W1, att2)
    a_src = a2[:, 0]
    a_dst = a2[:, 1]

    exr, den2 = _edge_w(a_src, a_dst, srcr, dstr)
    den3 = (den2[0, :N] + den2[1, :N]).reshape(N // RB, 1, RB)

    p1 = _spmm(xflat, srcr, dstr, exr)
    h2, x3 = _k2(p1, den3, W2, W2.T)

    p2 = _spmm(x3.reshape(NK * N, FC), srcr, dstr, exr)
    h4 = _k3(p2, den3, W1.T)
    return (h2, h4)
